# baseline (speedup 1.0000x reference)
import functools

import jax
import jax.numpy as jnp
from jax import lax
from jax.experimental import pallas as pl
from jax.experimental.pallas import tpu as pltpu
from jax.experimental.pallas import tpu_sc as plsc

N = 50000
NXv = NYv = NZv = 128
H = 0.015625
GMIN = -1.0
P = 125
P2 = 128
NB = 2048
NBLK = 25
NPAD = NB * NBLK
HALF = (NXv // 2) * NYv * NZv
TRASH = 8192
GBUF = HALF + TRASH
NS = 16
NC = 2
WROW = 512
WROWS = P2 * NPAD // WROW
RPT = WROWS // NS
RCH = 16
NCH = RPT // RCH


def _tc_body(means_ref, opac_ref, scales_ref, rot_ref,
             dens_ref, idx_ref, rad_ref):
    mx = means_ref[0:1, :]
    my = means_ref[1:2, :]
    mz = means_ref[2:3, :]
    op = opac_ref[0:1, :]
    sx = scales_ref[0:1, :]
    sy = scales_ref[1:2, :]
    sz = scales_ref[2:3, :]
    qw = rot_ref[0:1, :]
    qx = rot_ref[1:2, :]
    qy = rot_ref[2:3, :]
    qz = rot_ref[3:4, :]

    nrm = jnp.sqrt(qw * qw + qx * qx + qy * qy + qz * qz) + 1e-12
    w = qw / nrm
    x = qx / nrm
    y = qy / nrm
    z = qz / nrm

    r00 = 1 - 2 * (y * y + z * z)
    r01 = 2 * (x * y - w * z)
    r02 = 2 * (x * z + w * y)
    r10 = 2 * (x * y + w * z)
    r11 = 1 - 2 * (x * x + z * z)
    r12 = 2 * (y * z - w * x)
    r20 = 2 * (x * z - w * y)
    r21 = 2 * (y * z + w * x)
    r22 = 1 - 2 * (x * x + y * y)

    s1 = sx * sx
    s2 = sy * sy
    s3 = sz * sz
    a = r00 * r00 * s1 + r01 * r01 * s2 + r02 * r02 * s3 + 1e-6
    b = r00 * r10 * s1 + r01 * r11 * s2 + r02 * r12 * s3
    c = r00 * r20 * s1 + r01 * r21 * s2 + r02 * r22 * s3
    d = r10 * r10 * s1 + r11 * r11 * s2 + r12 * r12 * s3 + 1e-6
    e = r10 * r20 * s1 + r11 * r21 * s2 + r12 * r22 * s3
    f = r20 * r20 * s1 + r21 * r21 * s2 + r22 * r22 * s3 + 1e-6

    A00 = d * f - e * e
    A01 = c * e - b * f
    A02 = b * e - c * d
    A11 = a * f - c * c
    A12 = b * c - a * e
    A22 = a * d - b * b
    rdet = 1.0 / (a * A00 + b * A01 + c * A02)
    i00 = A00 * rdet
    i01 = A01 * rdet
    i02 = A02 * rdet
    i11 = A11 * rdet
    i12 = A12 * rdet
    i22 = A22 * rdet

    fx = jnp.floor((mx - GMIN) / H)
    fy = jnp.floor((my - GMIN) / H)
    fz = jnp.floor((mz - GMIN) / H)
    ix = fx.astype(jnp.int32)
    iy = fy.astype(jnp.int32)
    iz = fz.astype(jnp.int32)
    r0x = GMIN + (fx + 0.5) * H - mx
    r0y = GMIN + (fy + 0.5) * H - my
    r0z = GMIN + (fz + 0.5) * H - mz

    prow = lax.broadcasted_iota(jnp.int32, (P2, 1), 0)
    ox = prow // 25 - 2
    oy = (prow // 5) % 5 - 2
    oz = prow % 5 - 2
    dx = r0x + ox.astype(jnp.float32) * H
    dy = r0y + oy.astype(jnp.float32) * H
    dz = r0z + oz.astype(jnp.float32) * H
    qf = (i00 * dx * dx + i11 * dy * dy + i22 * dz * dz
          + 2.0 * (i01 * dx * dy + i02 * dx * dz + i12 * dy * dz))
    valid = prow < P
    dens_ref[...] = jnp.where(valid, op * jnp.exp(-0.5 * qf), 0.0)

    lin0 = ix * (NYv * NZv) + iy * NZv + iz
    shift = ox * (NYv * NZv) + oy * NZv + oz
    lin = lin0 + shift
    lane = lax.broadcasted_iota(jnp.int32, (1, NB), 1)
    tr = HALF + (lane & (TRASH - 1))
    in0 = lin < HALF
    idx_ref[0:P2, :] = jnp.where(in0 & valid, lin, tr)
    idx_ref[P2:2 * P2, :] = jnp.where(jnp.logical_not(in0) & valid,
                                      lin - HALF, tr)

    def _bf(v):
        return v.astype(jnp.bfloat16).astype(jnp.float32)

    def _sq(v):
        v = _bf(v)
        return v * v

    da = _sq(r00 * sx) + _sq(r01 * sy) + _sq(r02 * sz) + 1e-6
    dd = _sq(r10 * sx) + _sq(r11 * sy) + _sq(r12 * sz) + 1e-6
    df = _sq(r20 * sx) + _sq(r21 * sy) + _sq(r22 * sz) + 1e-6
    rx = jnp.ceil(3.0 * jnp.sqrt(jnp.maximum(da, 0.0)) / H).astype(jnp.int32)
    ry = jnp.ceil(3.0 * jnp.sqrt(jnp.maximum(dd, 0.0)) / H).astype(jnp.int32)
    rz = jnp.ceil(3.0 * jnp.sqrt(jnp.maximum(df, 0.0)) / H).astype(jnp.int32)
    rad_ref[...] = jnp.concatenate([rx, ry, rz], axis=0)


_tc_call = pl.pallas_call(
    _tc_body,
    grid=(NBLK,),
    in_specs=[
        pl.BlockSpec((3, NB), lambda i: (0, i)),
        pl.BlockSpec((1, NB), lambda i: (0, i)),
        pl.BlockSpec((3, NB), lambda i: (0, i)),
        pl.BlockSpec((4, NB), lambda i: (0, i)),
    ],
    out_specs=[
        pl.BlockSpec((P2, NB), lambda i: (0, i)),
        pl.BlockSpec((2 * P2, NB), lambda i: (0, i)),
        pl.BlockSpec((3, NB), lambda i: (0, i)),
    ],
    out_shape=[
        jax.ShapeDtypeStruct((P2, NPAD), jnp.float32),
        jax.ShapeDtypeStruct((2 * P2, NPAD), jnp.int32),
        jax.ShapeDtypeStruct((3, NPAD), jnp.int32),
    ],
)


def _sc_scatter_body(dens_hbm, idx_hbm, zero_hbm, out_hbm, vals_v, idx_v,
                     grid_sh, scat_sem, load_sem):
    c = lax.axis_index("c")
    s = lax.axis_index("s")

    zch = GBUF // NS
    pltpu.sync_copy(zero_hbm.at[pl.ds(s * zch, zch)],
                    grid_sh.at[pl.ds(s * zch, zch)])
    plsc.subcore_barrier()

    base0 = s * RPT * WROW
    CW = RCH * WROW

    def fire_load(kk, b):
        r = base0 + kk * CW
        pltpu.async_copy(dens_hbm.at[pl.ds(r, CW)],
                         vals_v.at[pl.ds(b * CW, CW)], load_sem)
        pltpu.async_copy(idx_hbm.at[c, pl.ds(r, CW)],
                         idx_v.at[pl.ds(b * CW, CW)], load_sem)

    def wait_load(kk, b):
        r = base0 + kk * CW
        pltpu.make_async_copy(dens_hbm.at[pl.ds(r, CW)],
                              vals_v.at[pl.ds(b * CW, CW)], load_sem).wait()
        pltpu.make_async_copy(idx_hbm.at[c, pl.ds(r, CW)],
                              idx_v.at[pl.ds(b * CW, CW)], load_sem).wait()

    fire_load(0, 0)

    def chunk_body(k, carry):
        b = k & 1
        wait_load(k, b)

        @pl.when(k + 1 < NCH)
        def _prefetch():
            fire_load(k + 1, 1 - b)

        cps = [pltpu.async_copy(
                   vals_v.at[pl.ds(b * CW + j * WROW, WROW)],
                   grid_sh.at[idx_v.at[pl.ds(b * CW + j * WROW, WROW)]],
                   scat_sem, add=True)
               for j in range(RCH)]
        for cp in cps:
            cp.wait()
        return carry

    lax.fori_loop(0, NCH, chunk_body, 0)
    plsc.subcore_barrier()

    hch = HALF // NS
    pltpu.sync_copy(grid_sh.at[pl.ds(s * hch, hch)],
                    out_hbm.at[pl.ds(c * HALF + s * hch, hch)])


@functools.lru_cache(maxsize=1)
def _sc_scatter():
    return pl.kernel(
        _sc_scatter_body,
        out_type=jax.ShapeDtypeStruct((2 * HALF,), jnp.float32),
        mesh=plsc.VectorSubcoreMesh(core_axis_name="c", subcore_axis_name="s",
                                    num_cores=NC, num_subcores=NS),
        scratch_types=[
            pltpu.VMEM((2 * RCH * WROW,), jnp.float32),
            pltpu.VMEM((2 * RCH * WROW,), jnp.int32),
            pltpu.VMEM_SHARED((GBUF,), jnp.float32),
            pltpu.SemaphoreType.DMA,
            pltpu.SemaphoreType.DMA,
        ],
    )


def kernel(means3D, opacities, scales, rotations):
    f32 = jnp.float32
    pad = NPAD - N
    meansT = jnp.concatenate(
        [means3D.astype(f32), jnp.zeros((pad, 3), f32)], axis=0).T
    opacT = jnp.concatenate(
        [opacities.astype(f32), jnp.zeros((pad, 1), f32)], axis=0).T
    scalesT = jnp.concatenate(
        [scales.astype(f32), jnp.full((pad, 3), 0.01, f32)], axis=0).T
    rotT = jnp.concatenate(
        [rotations.astype(f32),
         jnp.tile(jnp.array([[1.0, 0.0, 0.0, 0.0]], f32), (pad, 1))],
        axis=0).T

    dens, idx, rad = _tc_call(meansT, opacT, scalesT, rotT)
    zeros = jnp.zeros((GBUF,), f32)
    dens_w = dens.reshape(WROWS * WROW)
    idx_w = idx.reshape(2, WROWS * WROW)
    flat = _sc_scatter()(dens_w, idx_w, zeros)
    fields = flat.reshape(NXv, NYv, NZv)
    return (fields, rad[0, :N], rad[1, :N], rad[2, :N])


# speedup vs baseline: 1.2034x; 1.2034x over previous
import functools

import jax
import jax.numpy as jnp
from jax import lax
from jax.experimental import pallas as pl
from jax.experimental.pallas import tpu as pltpu
from jax.experimental.pallas import tpu_sc as plsc

N = 50000
NXv = NYv = NZv = 128
H = 0.015625
GMIN = -1.0
P = 125
P2 = 128
NB = 2048
NBLK = 25
NPAD = NB * NBLK
HALF = (NXv // 2) * NYv * NZv
TRASH = 8192
GBUF = HALF + TRASH
NS = 16
NC = 2
COLS_PER_TILE = NPAD // NS
CHUNK = 128
NCHUNK = COLS_PER_TILE // CHUNK


def _tc_body(means_ref, opac_ref, scales_ref, rot_ref,
             dens_ref, idx_ref, rad_ref):
    mx = means_ref[0:1, :]
    my = means_ref[1:2, :]
    mz = means_ref[2:3, :]
    op = opac_ref[0:1, :]
    sx = scales_ref[0:1, :]
    sy = scales_ref[1:2, :]
    sz = scales_ref[2:3, :]
    qw = rot_ref[0:1, :]
    qx = rot_ref[1:2, :]
    qy = rot_ref[2:3, :]
    qz = rot_ref[3:4, :]

    nrm = jnp.sqrt(qw * qw + qx * qx + qy * qy + qz * qz) + 1e-12
    w = qw / nrm
    x = qx / nrm
    y = qy / nrm
    z = qz / nrm

    r00 = 1 - 2 * (y * y + z * z)
    r01 = 2 * (x * y - w * z)
    r02 = 2 * (x * z + w * y)
    r10 = 2 * (x * y + w * z)
    r11 = 1 - 2 * (x * x + z * z)
    r12 = 2 * (y * z - w * x)
    r20 = 2 * (x * z - w * y)
    r21 = 2 * (y * z + w * x)
    r22 = 1 - 2 * (x * x + y * y)

    s1 = sx * sx
    s2 = sy * sy
    s3 = sz * sz
    a = r00 * r00 * s1 + r01 * r01 * s2 + r02 * r02 * s3 + 1e-6
    b = r00 * r10 * s1 + r01 * r11 * s2 + r02 * r12 * s3
    c = r00 * r20 * s1 + r01 * r21 * s2 + r02 * r22 * s3
    d = r10 * r10 * s1 + r11 * r11 * s2 + r12 * r12 * s3 + 1e-6
    e = r10 * r20 * s1 + r11 * r21 * s2 + r12 * r22 * s3
    f = r20 * r20 * s1 + r21 * r21 * s2 + r22 * r22 * s3 + 1e-6

    A00 = d * f - e * e
    A01 = c * e - b * f
    A02 = b * e - c * d
    A11 = a * f - c * c
    A12 = b * c - a * e
    A22 = a * d - b * b
    rdet = 1.0 / (a * A00 + b * A01 + c * A02)
    i00 = A00 * rdet
    i01 = A01 * rdet
    i02 = A02 * rdet
    i11 = A11 * rdet
    i12 = A12 * rdet
    i22 = A22 * rdet

    fx = jnp.floor((mx - GMIN) / H)
    fy = jnp.floor((my - GMIN) / H)
    fz = jnp.floor((mz - GMIN) / H)
    ix = fx.astype(jnp.int32)
    iy = fy.astype(jnp.int32)
    iz = fz.astype(jnp.int32)
    r0x = GMIN + (fx + 0.5) * H - mx
    r0y = GMIN + (fy + 0.5) * H - my
    r0z = GMIN + (fz + 0.5) * H - mz

    prow = lax.broadcasted_iota(jnp.int32, (P2, 1), 0)
    ox = prow // 25 - 2
    oy = (prow // 5) % 5 - 2
    oz = prow % 5 - 2
    dx = r0x + ox.astype(jnp.float32) * H
    dy = r0y + oy.astype(jnp.float32) * H
    dz = r0z + oz.astype(jnp.float32) * H
    qf = (i00 * dx * dx + i11 * dy * dy + i22 * dz * dz
          + 2.0 * (i01 * dx * dy + i02 * dx * dz + i12 * dy * dz))
    valid = prow < P
    dens_ref[...] = jnp.where(valid, op * jnp.exp(-0.5 * qf), 0.0)

    lin0 = ix * (NYv * NZv) + iy * NZv + iz
    shift = ox * (NYv * NZv) + oy * NZv + oz
    lin = lin0 + shift
    lane = lax.broadcasted_iota(jnp.int32, (1, NB), 1)
    tr = HALF + (lane & (TRASH - 1))
    in0 = lin < HALF
    idx_ref[0:P2, :] = jnp.where(in0 & valid, lin, tr)
    idx_ref[P2:2 * P2, :] = jnp.where(jnp.logical_not(in0) & valid,
                                      lin - HALF, tr)

    def _bf(v):
        return v.astype(jnp.bfloat16).astype(jnp.float32)

    def _sq(v):
        v = _bf(v)
        return v * v

    da = _sq(r00 * sx) + _sq(r01 * sy) + _sq(r02 * sz) + 1e-6
    dd = _sq(r10 * sx) + _sq(r11 * sy) + _sq(r12 * sz) + 1e-6
    df = _sq(r20 * sx) + _sq(r21 * sy) + _sq(r22 * sz) + 1e-6
    rx = jnp.ceil(3.0 * jnp.sqrt(jnp.maximum(da, 0.0)) / H).astype(jnp.int32)
    ry = jnp.ceil(3.0 * jnp.sqrt(jnp.maximum(dd, 0.0)) / H).astype(jnp.int32)
    rz = jnp.ceil(3.0 * jnp.sqrt(jnp.maximum(df, 0.0)) / H).astype(jnp.int32)
    rad_ref[...] = jnp.concatenate([rx, ry, rz], axis=0)


_tc_call = pl.pallas_call(
    _tc_body,
    grid=(NBLK,),
    in_specs=[
        pl.BlockSpec((3, NB), lambda i: (0, i)),
        pl.BlockSpec((1, NB), lambda i: (0, i)),
        pl.BlockSpec((3, NB), lambda i: (0, i)),
        pl.BlockSpec((4, NB), lambda i: (0, i)),
    ],
    out_specs=[
        pl.BlockSpec((P2, NB), lambda i: (0, i)),
        pl.BlockSpec((2 * P2, NB), lambda i: (0, i)),
        pl.BlockSpec((3, NB), lambda i: (0, i)),
    ],
    out_shape=[
        jax.ShapeDtypeStruct((P2, NPAD), jnp.float32),
        jax.ShapeDtypeStruct((2 * P2, NPAD), jnp.int32),
        jax.ShapeDtypeStruct((3, NPAD), jnp.int32),
    ],
)


def _sc_scatter_body(dens_hbm, idx_hbm, zero_hbm, out_hbm, vals_v, idx_v,
                     grid_sh, scat_sem, load_sem):
    c = lax.axis_index("c")
    s = lax.axis_index("s")

    zch = GBUF // NS
    pltpu.sync_copy(zero_hbm.at[pl.ds(s * zch, zch)],
                    grid_sh.at[pl.ds(s * zch, zch)])
    plsc.subcore_barrier()

    col0 = s * COLS_PER_TILE
    crow = c * P2
    GRP = 16
    HR = 64
    NCH2 = NCHUNK * 2

    def fire_load(kk, b):
        col = col0 + (kk // 2) * CHUNK
        row = (kk % 2) * HR
        pltpu.async_copy(dens_hbm.at[pl.ds(row, HR), pl.ds(col, CHUNK)],
                         vals_v.at[pl.ds(b * HR, HR)], load_sem)
        pltpu.async_copy(idx_hbm.at[pl.ds(crow + row, HR), pl.ds(col, CHUNK)],
                         idx_v.at[pl.ds(b * HR, HR)], load_sem)

    def wait_load(kk, b):
        col = col0 + (kk // 2) * CHUNK
        row = (kk % 2) * HR
        pltpu.make_async_copy(dens_hbm.at[pl.ds(row, HR), pl.ds(col, CHUNK)],
                              vals_v.at[pl.ds(b * HR, HR)], load_sem).wait()
        pltpu.make_async_copy(idx_hbm.at[pl.ds(crow + row, HR),
                                         pl.ds(col, CHUNK)],
                              idx_v.at[pl.ds(b * HR, HR)], load_sem).wait()

    fire_load(0, 0)

    def chunk_body(k, carry):
        b = k & 1
        wait_load(k, b)

        @pl.when(k + 1 < NCH2)
        def _prefetch():
            fire_load(k + 1, 1 - b)

        def group_body(g, carry2):
            base = b * HR + g * GRP
            cps = [pltpu.async_copy(vals_v.at[base + j],
                                    grid_sh.at[idx_v.at[base + j]],
                                    scat_sem, add=True)
                   for j in range(GRP)]
            for cp in cps:
                cp.wait()
            return carry2

        return lax.fori_loop(0, HR // GRP, group_body, carry)

    lax.fori_loop(0, NCH2, chunk_body, 0)
    plsc.subcore_barrier()

    hch = HALF // NS
    pltpu.sync_copy(grid_sh.at[pl.ds(s * hch, hch)],
                    out_hbm.at[pl.ds(c * HALF + s * hch, hch)])


@functools.lru_cache(maxsize=1)
def _sc_scatter():
    return pl.kernel(
        _sc_scatter_body,
        out_type=jax.ShapeDtypeStruct((2 * HALF,), jnp.float32),
        mesh=plsc.VectorSubcoreMesh(core_axis_name="c", subcore_axis_name="s",
                                    num_cores=NC, num_subcores=NS),
        scratch_types=[
            pltpu.VMEM((128, CHUNK), jnp.float32),
            pltpu.VMEM((128, CHUNK), jnp.int32),
            pltpu.VMEM_SHARED((GBUF,), jnp.float32),
            pltpu.SemaphoreType.DMA,
            pltpu.SemaphoreType.DMA,
        ],
    )


def kernel(means3D, opacities, scales, rotations):
    f32 = jnp.float32
    pad = NPAD - N
    meansT = jnp.concatenate(
        [means3D.astype(f32), jnp.zeros((pad, 3), f32)], axis=0).T
    opacT = jnp.concatenate(
        [opacities.astype(f32), jnp.zeros((pad, 1), f32)], axis=0).T
    scalesT = jnp.concatenate(
        [scales.astype(f32), jnp.full((pad, 3), 0.01, f32)], axis=0).T
    rotT = jnp.concatenate(
        [rotations.astype(f32),
         jnp.tile(jnp.array([[1.0, 0.0, 0.0, 0.0]], f32), (pad, 1))],
        axis=0).T

    dens, idx, rad = _tc_call(meansT, opacT, scalesT, rotT)
    zeros = jnp.zeros((GBUF,), f32)
    flat = _sc_scatter()(dens, idx, zeros)
    fields = flat.reshape(NXv, NYv, NZv)
    return (fields, rad[0, :N], rad[1, :N], rad[2, :N])
